# baseline (speedup 1.0000x reference)
import functools

import jax
import jax.numpy as jnp
from jax import lax
from jax.experimental import pallas as pl
from jax.experimental.pallas import tpu as pltpu
from jax.experimental.pallas import tpu_sc as plsc

D_MODEL = 768
BATCH = 4
SEQ = 2048
NB = BATCH * SEQ
NW = 32
B_PER_W = NB // NW
K = 32
N_CHUNKS = B_PER_W // K
LANES = 16
JCOLS = D_MODEL // LANES


def _make_embed():
    mesh = plsc.VectorSubcoreMesh(core_axis_name="c", subcore_axis_name="s")

    @functools.partial(
        pl.kernel,
        mesh=mesh,
        out_type=jax.ShapeDtypeStruct((NB, D_MODEL), jnp.float32),
        scratch_types=[
            pltpu.VMEM((B_PER_W,), jnp.int32),
            pltpu.VMEM((K, D_MODEL), jnp.float32),
            pltpu.VMEM((K, D_MODEL), jnp.float32),
            pltpu.SemaphoreType.DMA,
        ],
    )
    def embed(x_hbm, table_hbm, pos_hbm, out_hbm, idx_v, pos_v, rows_v, sem):
        wid = lax.axis_index("s") * 2 + lax.axis_index("c")
        base = pl.multiple_of(wid * B_PER_W, B_PER_W)
        pos_base = pl.multiple_of(lax.rem(base, SEQ), B_PER_W)
        pltpu.sync_copy(x_hbm.at[pl.ds(base, B_PER_W)], idx_v)

        def chunk(c, carry):
            off = pl.multiple_of(c * K, K)
            gather = pltpu.async_copy(
                table_hbm.at[idx_v.at[pl.ds(off, K)]], rows_v, sem)
            pltpu.sync_copy(pos_hbm.at[pl.ds(pos_base + off, K)], pos_v)
            gather.wait()

            def row(r, rcarry):
                for j in range(JCOLS):
                    sl = pl.ds(j * LANES, LANES)
                    rows_v[r, sl] = rows_v[r, sl] + pos_v[r, sl]
                return rcarry

            lax.fori_loop(0, K, row, 0)
            pltpu.sync_copy(rows_v, out_hbm.at[pl.ds(base + off, K)])
            return carry

        lax.fori_loop(0, N_CHUNKS, chunk, 0)

    return embed


_embed = _make_embed()


def kernel(x, token_table, pos_emb):
    x_flat = x.reshape(NB)
    pos = pos_emb.reshape(pos_emb.shape[1], D_MODEL)
    out = _embed(x_flat, token_table, pos)
    return out.reshape(BATCH, SEQ, D_MODEL)
